# baseline (speedup 1.0000x reference)
import jax
import jax.numpy as jnp
from jax.experimental import pallas as pl
from jax.experimental.pallas import tpu as pltpu

_ALPHA = 1e-05
_NF = 128


def _mxu_t(x, m):
    eye = (jax.lax.broadcasted_iota(jnp.int32, (m, m), 0) ==
           jax.lax.broadcasted_iota(jnp.int32, (m, m), 1)).astype(jnp.float32)
    return jax.lax.dot_general(eye, x, (((0,), (1,)), ((), ())),
                               preferred_element_type=jnp.float32)


def _sampler_body(t0_ref, t1_ref, ts_ref, w_ref, u_ref, tr_ref, out_ref):
    rblk, nc = ts_ref.shape
    ts = _mxu_t(ts_ref[...], nc)
    w = _mxu_t(w_ref[...], nc) + _ALPHA
    t0 = t0_ref[...]
    t1 = t1_ref[...]

    row = jax.lax.broadcasted_iota(jnp.int32, (nc, nc), 0)
    col = jax.lax.broadcasted_iota(jnp.int32, (nc, nc), 1)
    tri = (col <= row).astype(jnp.float32)
    cdf = jax.lax.dot(tri, w, preferred_element_type=jnp.float32)
    total = cdf[nc - 1:nc, :]

    mids = (ts[1:, :] + ts[:-1, :]) * 0.5
    lo = jnp.concatenate([t0, mids], axis=0)
    up = jnp.concatenate([mids, t1], axis=0)
    wid = up - lo
    zrow = jnp.zeros_like(t0)
    dlo = jnp.concatenate([lo[1:, :] - lo[:-1, :], zrow], axis=0)
    dwid = jnp.concatenate([wid[1:, :] - wid[:-1, :], zrow], axis=0)

    schunk = 32
    rsub = min(128, rblk)
    rchunk = min(64, rblk)
    lane = jax.lax.broadcasted_iota(jnp.int32, (rchunk, _NF), 1)

    for rb in range(0, rblk, rsub):
        uqt = _mxu_t(u_ref[rb:rb + rsub, :], _NF) * total[:, rb:rb + rsub]
        lo_parts, wid_parts = [], []
        for c in range(0, _NF, schunk):
            uq_c = uqt[c:c + schunk, :]
            acc_lo = jnp.broadcast_to(lo[0:1, rb:rb + rsub], (schunk, rsub))
            acc_wid = jnp.broadcast_to(wid[0:1, rb:rb + rsub], (schunk, rsub))
            for j in range(nc):
                m = uq_c > cdf[j:j + 1, rb:rb + rsub]
                acc_lo = acc_lo + jnp.where(m, dlo[j:j + 1, rb:rb + rsub], 0.0)
                acc_wid = acc_wid + jnp.where(m, dwid[j:j + 1, rb:rb + rsub], 0.0)
            lo_parts.append(acc_lo)
            wid_parts.append(acc_wid)
        lo_s = _mxu_t(jnp.concatenate(lo_parts, axis=0), rsub)
        wid_s = _mxu_t(jnp.concatenate(wid_parts, axis=0), rsub)
        v = lo_s + wid_s * tr_ref[rb:rb + rsub, :]

        for rc in range(0, rsub, rchunk):
            vc = v[rc:rc + rchunk, :]
            k = 2
            while k <= _NF:
                d = k // 2
                while d >= 1:
                    partner = jnp.where((lane & d) == 0,
                                        pltpu.roll(vc, _NF - d, 1),
                                        pltpu.roll(vc, d, 1))
                    take_min = ((lane & k) == 0) == ((lane & d) == 0)
                    vc = jnp.where(take_min,
                                   jnp.minimum(vc, partner),
                                   jnp.maximum(vc, partner))
                    d //= 2
                k *= 2
            out_ref[rb + rc:rb + rc + rchunk, :] = vc


def kernel(t0, t1, ts, weights, n_samples):
    del n_samples
    b, r, nc, _ = ts.shape
    n = b * r
    rblk = min(256, n)

    t0r = t0.reshape(1, n)
    t1r = t1.reshape(1, n)
    tsr = ts.reshape(n, nc)
    wr = weights.reshape(n, nc)
    u = jax.random.uniform(jax.random.key(1), (n, _NF), dtype=weights.dtype)
    tr = jax.random.uniform(jax.random.key(2), (b, r, _NF, 1),
                            dtype=ts.dtype).reshape(n, _NF)

    out = pl.pallas_call(
        _sampler_body,
        grid=(n // rblk,),
        in_specs=[
            pl.BlockSpec((1, rblk), lambda i: (0, i)),
            pl.BlockSpec((1, rblk), lambda i: (0, i)),
            pl.BlockSpec((rblk, nc), lambda i: (i, 0)),
            pl.BlockSpec((rblk, nc), lambda i: (i, 0)),
            pl.BlockSpec((rblk, _NF), lambda i: (i, 0)),
            pl.BlockSpec((rblk, _NF), lambda i: (i, 0)),
        ],
        out_specs=pl.BlockSpec((rblk, _NF), lambda i: (i, 0)),
        out_shape=jax.ShapeDtypeStruct((n, _NF), jnp.float32),
        compiler_params=pltpu.CompilerParams(
            dimension_semantics=("arbitrary",),
        ),
    )(t0r, t1r, tsr, wr, u, tr)
    return out.reshape(b, r, _NF, 1)
